# baseline (speedup 1.0000x reference)
import functools

import jax
import jax.numpy as jnp
from jax import lax
from jax.experimental import pallas as pl
from jax.experimental.pallas import tpu as pltpu
from jax.experimental.pallas import tpu_sc as plsc

_N = 16384
_P_DIM = 64
_Q_DIM = 64
_NUM_WORKERS = 32
_TC_BLOCK = 512
_N_CHUNKS = 4
_CHUNK = _N // _N_CHUNKS


def _sc_gather(x0, v0, d, w, n_rows):
    rpw = n_rows // _NUM_WORKERS
    mesh = plsc.VectorSubcoreMesh(core_axis_name="c", subcore_axis_name="s")

    @functools.partial(
        pl.kernel,
        out_type=[
            jax.ShapeDtypeStruct((n_rows, _P_DIM), jnp.float32),
            jax.ShapeDtypeStruct((n_rows, _Q_DIM), jnp.float32),
        ],
        mesh=mesh,
        scratch_types=[
            pltpu.VMEM((rpw,), jnp.int32),
            pltpu.VMEM((rpw,), jnp.int32),
            pltpu.VMEM((rpw, _P_DIM), jnp.float32),
            pltpu.VMEM((rpw, _Q_DIM), jnp.float32),
            pltpu.SemaphoreType.DMA,
            pltpu.SemaphoreType.DMA,
        ],
        compiler_params=pltpu.CompilerParams(use_tc_tiling_on_sc=False),
    )
    def gather_kernel(x0_hbm, v0_hbm, d_hbm, w_hbm, outx_hbm, outw_hbm,
                      idx_d, idx_w, rows_x, rows_w, sem_x, sem_w):
        wid = lax.axis_index("s") * 2 + lax.axis_index("c")
        base = wid * rpw
        pltpu.sync_copy(d_hbm.at[pl.ds(base, rpw)], idx_d)
        pltpu.sync_copy(w_hbm.at[pl.ds(base, rpw)], idx_w)
        cx = pltpu.async_copy(x0_hbm.at[idx_d], rows_x, sem_x)
        cw = pltpu.async_copy(v0_hbm.at[idx_w], rows_w, sem_w)
        cx.wait()
        cw.wait()
        pltpu.sync_copy(rows_x, outx_hbm.at[pl.ds(base, rpw)])
        pltpu.sync_copy(rows_w, outw_hbm.at[pl.ds(base, rpw)])

    return gather_kernel(x0, v0, d, w)


def _tc_expand_body(x_ref, w_ref, r_ref, o_ref):
    x = x_ref[...]
    w = w_ref[...]
    sx = jnp.sum(x * x, axis=1, keepdims=True)
    sw = jnp.sum(w * w, axis=1, keepdims=True)
    xs = x * lax.rsqrt(sx * sw)
    xrep = jnp.dot(xs, r_ref[...], preferred_element_type=jnp.float32)
    wtile = pltpu.repeat(w, _P_DIM, axis=1)
    o_ref[...] = xrep * wtile


def _rmat():
    jm = jnp.arange(_P_DIM * _Q_DIM, dtype=jnp.int32) // _Q_DIM
    rmat = (jm[None, :] == jnp.arange(_P_DIM, dtype=jnp.int32)[:, None])
    return rmat.astype(jnp.float32)


def _tc_expand_chunk(rows_x, rows_w, rmat, prev, c):
    steps = _CHUNK // _TC_BLOCK
    base = c * steps

    def body(x_ref, w_ref, r_ref, prev_ref, o_ref):
        del prev_ref
        _tc_expand_body(x_ref, w_ref, r_ref, o_ref)

    return pl.pallas_call(
        body,
        grid=(steps,),
        in_specs=[
            pl.BlockSpec((_TC_BLOCK, _P_DIM), lambda i: (i, 0)),
            pl.BlockSpec((_TC_BLOCK, _Q_DIM), lambda i: (i, 0)),
            pl.BlockSpec((_P_DIM, _P_DIM * _Q_DIM), lambda i: (0, 0)),
            pl.BlockSpec(memory_space=pl.ANY),
        ],
        out_specs=pl.BlockSpec((_TC_BLOCK, _P_DIM * _Q_DIM),
                               lambda i, base=base: (base + i, 0)),
        out_shape=jax.ShapeDtypeStruct((_N, _P_DIM * _Q_DIM), jnp.float32),
        input_output_aliases={3: 0},
        compiler_params=pltpu.CompilerParams(
            dimension_semantics=("arbitrary",),
        ),
    )(rows_x, rows_w, rmat, prev)


def _tc_expand_first(rows_x, rows_w, rmat):
    steps = _CHUNK // _TC_BLOCK
    return pl.pallas_call(
        _tc_expand_body,
        grid=(steps,),
        in_specs=[
            pl.BlockSpec((_TC_BLOCK, _P_DIM), lambda i: (i, 0)),
            pl.BlockSpec((_TC_BLOCK, _Q_DIM), lambda i: (i, 0)),
            pl.BlockSpec((_P_DIM, _P_DIM * _Q_DIM), lambda i: (0, 0)),
        ],
        out_specs=pl.BlockSpec((_TC_BLOCK, _P_DIM * _Q_DIM), lambda i: (i, 0)),
        out_shape=jax.ShapeDtypeStruct((_N, _P_DIM * _Q_DIM), jnp.float32),
        compiler_params=pltpu.CompilerParams(
            dimension_semantics=("arbitrary",),
        ),
    )(rows_x, rows_w, rmat)


@jax.jit
def kernel(d, w, x0, v0):
    rmat = _rmat()
    chunks = []
    for c in range(_N_CHUNKS):
        dc = lax.slice_in_dim(d, c * _CHUNK, (c + 1) * _CHUNK)
        wc = lax.slice_in_dim(w, c * _CHUNK, (c + 1) * _CHUNK)
        chunks.append(_sc_gather(x0, v0, dc, wc, _CHUNK))
    out = _tc_expand_first(chunks[0][0], chunks[0][1], rmat)
    for c in range(1, _N_CHUNKS):
        out = _tc_expand_chunk(chunks[c][0], chunks[c][1], rmat, out, c)
    return out


# speedup vs baseline: 1.0837x; 1.0837x over previous
import functools

import jax
import jax.numpy as jnp
from jax import lax
from jax.experimental import pallas as pl
from jax.experimental.pallas import tpu as pltpu
from jax.experimental.pallas import tpu_sc as plsc

_N = 16384
_P_DIM = 64
_Q_DIM = 64
_NUM_WORKERS = 32
_TC_BLOCK = 512
_N_CHUNKS = 4
_CHUNK = _N // _N_CHUNKS


def _sc_gather(x0, v0, d, w, n_rows):
    rpw = n_rows // _NUM_WORKERS
    mesh = plsc.VectorSubcoreMesh(core_axis_name="c", subcore_axis_name="s")

    @functools.partial(
        pl.kernel,
        out_type=[
            jax.ShapeDtypeStruct((n_rows, _P_DIM), jnp.float32),
            jax.ShapeDtypeStruct((n_rows, _Q_DIM), jnp.float32),
        ],
        mesh=mesh,
        scratch_types=[
            pltpu.VMEM((rpw,), jnp.int32),
            pltpu.VMEM((rpw,), jnp.int32),
            pltpu.VMEM((rpw, _P_DIM), jnp.float32),
            pltpu.VMEM((rpw, _Q_DIM), jnp.float32),
            pltpu.SemaphoreType.DMA,
            pltpu.SemaphoreType.DMA,
        ],
        compiler_params=pltpu.CompilerParams(use_tc_tiling_on_sc=False),
    )
    def gather_kernel(x0_hbm, v0_hbm, d_hbm, w_hbm, outx_hbm, outw_hbm,
                      idx_d, idx_w, rows_x, rows_w, sem_x, sem_w):
        wid = lax.axis_index("s") * 2 + lax.axis_index("c")
        base = wid * rpw
        pltpu.sync_copy(d_hbm.at[pl.ds(base, rpw)], idx_d)
        pltpu.sync_copy(w_hbm.at[pl.ds(base, rpw)], idx_w)
        cx = pltpu.async_copy(x0_hbm.at[idx_d], rows_x, sem_x)
        cw = pltpu.async_copy(v0_hbm.at[idx_w], rows_w, sem_w)
        cx.wait()
        cw.wait()
        pltpu.sync_copy(rows_x, outx_hbm.at[pl.ds(base, rpw)])
        pltpu.sync_copy(rows_w, outw_hbm.at[pl.ds(base, rpw)])

    return gather_kernel(x0, v0, d, w)


def _tc_expand_body(x_ref, w_ref, r_ref, o_ref):
    x = x_ref[...]
    w = w_ref[...]
    sx = jnp.sum(x * x, axis=1, keepdims=True)
    sw = jnp.sum(w * w, axis=1, keepdims=True)
    xs = x * lax.rsqrt(sx * sw)
    xrep = jnp.dot(xs, r_ref[...], preferred_element_type=jnp.float32)
    wtile = pltpu.repeat(w, _P_DIM, axis=1)
    o_ref[...] = xrep * wtile


def _rmat():
    jm = jnp.arange(_P_DIM * _Q_DIM, dtype=jnp.int32) // _Q_DIM
    rmat = (jm[None, :] == jnp.arange(_P_DIM, dtype=jnp.int32)[:, None])
    return rmat.astype(jnp.float32)


_NQ = 4


def _tc_expand_manual_body(x_hbm, w_hbm, r_hbm, o_hbm,
                           xv, wv, rv, ob, insem, outsem, rsem):
    nb = _N // _TC_BLOCK
    rcp = pltpu.make_async_copy(r_hbm, rv, rsem)
    rcp.start()

    def start_inputs(i):
        sl = pl.ds(i * _TC_BLOCK, _TC_BLOCK)
        cx = pltpu.make_async_copy(x_hbm.at[sl], xv.at[i % 2], insem.at[i % 2])
        cw = pltpu.make_async_copy(w_hbm.at[sl], wv.at[i % 2], insem.at[i % 2])
        cx.start()
        cw.start()
        return (cx, cw)

    pending_in = start_inputs(0)
    rcp.wait()
    out_copies = [None] * nb
    for i in range(nb):
        cx, cw = pending_in
        cx.wait()
        cw.wait()
        if i + 1 < nb:
            pending_in = start_inputs(i + 1)
        if i >= _NQ:
            out_copies[i - _NQ].wait()
        x = xv[i % 2]
        w = wv[i % 2]
        sx = jnp.sum(x * x, axis=1, keepdims=True)
        sw = jnp.sum(w * w, axis=1, keepdims=True)
        xs = x * lax.rsqrt(sx * sw)
        xrep = jnp.dot(xs, rv[...], preferred_element_type=jnp.float32)
        wtile = pltpu.repeat(w, _P_DIM, axis=1)
        ob[i % _NQ] = xrep * wtile
        co = pltpu.make_async_copy(
            ob.at[i % _NQ], o_hbm.at[pl.ds(i * _TC_BLOCK, _TC_BLOCK)],
            outsem.at[i % _NQ])
        co.start()
        out_copies[i] = co
    for i in range(nb - _NQ, nb):
        out_copies[i].wait()


def _tc_expand(rows_x, rows_w, rmat):
    return pl.pallas_call(
        _tc_expand_manual_body,
        in_specs=[
            pl.BlockSpec(memory_space=pl.ANY),
            pl.BlockSpec(memory_space=pl.ANY),
            pl.BlockSpec(memory_space=pl.ANY),
        ],
        out_specs=pl.BlockSpec(memory_space=pl.ANY),
        out_shape=jax.ShapeDtypeStruct((_N, _P_DIM * _Q_DIM), jnp.float32),
        scratch_shapes=[
            pltpu.VMEM((2, _TC_BLOCK, _P_DIM), jnp.float32),
            pltpu.VMEM((2, _TC_BLOCK, _Q_DIM), jnp.float32),
            pltpu.VMEM((_P_DIM, _P_DIM * _Q_DIM), jnp.float32),
            pltpu.VMEM((_NQ, _TC_BLOCK, _P_DIM * _Q_DIM), jnp.float32),
            pltpu.SemaphoreType.DMA((2,)),
            pltpu.SemaphoreType.DMA((_NQ,)),
            pltpu.SemaphoreType.DMA,
        ],
        compiler_params=pltpu.CompilerParams(
            vmem_limit_bytes=56 * 1024 * 1024,
        ),
    )(rows_x, rows_w, rmat)


@jax.jit
def kernel(d, w, x0, v0):
    rows_x, rows_w = _sc_gather(x0, v0, d, w, _N)
    return _tc_expand(rows_x, rows_w, _rmat())


# speedup vs baseline: 1.2098x; 1.1164x over previous
import functools

import jax
import jax.numpy as jnp
from jax import lax
from jax.experimental import pallas as pl
from jax.experimental.pallas import tpu as pltpu
from jax.experimental.pallas import tpu_sc as plsc

_N = 16384
_P_DIM = 64
_Q_DIM = 64
_NUM_WORKERS = 32
_RPW = _N // _NUM_WORKERS
_TC_BLOCK = 1024


def _sc_gather(x0, d):
    mesh = plsc.VectorSubcoreMesh(core_axis_name="c", subcore_axis_name="s")

    @functools.partial(
        pl.kernel,
        out_type=jax.ShapeDtypeStruct((_N, _P_DIM), jnp.float32),
        mesh=mesh,
        scratch_types=[
            pltpu.VMEM((_RPW,), jnp.int32),
            pltpu.VMEM((_RPW, _P_DIM), jnp.float32),
            pltpu.SemaphoreType.DMA,
        ],
        compiler_params=pltpu.CompilerParams(use_tc_tiling_on_sc=False),
    )
    def gather_kernel(x0_hbm, d_hbm, outx_hbm, idx_d, rows_x, sem_x):
        wid = lax.axis_index("s") * 2 + lax.axis_index("c")
        base = wid * _RPW
        pltpu.sync_copy(d_hbm.at[pl.ds(base, _RPW)], idx_d)
        pltpu.async_copy(x0_hbm.at[idx_d], rows_x, sem_x).wait()
        pltpu.sync_copy(rows_x, outx_hbm.at[pl.ds(base, _RPW)])

    return gather_kernel(x0, d)


def _tc_expand_body(x_ref, w_ref, v_ref, r_ref, o_ref):
    x = x_ref[...]
    wi = w_ref[...]
    v = v_ref[...]
    sv = jnp.sum(v * v, axis=1, keepdims=True)
    vn = v * lax.rsqrt(sv)
    oh = (lax.broadcasted_iota(jnp.int32, (x.shape[0], _Q_DIM), 1) == wi)
    w_rows = jnp.dot(oh.astype(jnp.float32), vn,
                     preferred_element_type=jnp.float32)
    sx = jnp.sum(x * x, axis=1, keepdims=True)
    xs = x * lax.rsqrt(sx)
    xrep = jnp.dot(xs, r_ref[...], preferred_element_type=jnp.float32)
    wtile = pltpu.repeat(w_rows, _P_DIM, axis=1)
    o_ref[...] = xrep * wtile


def _rmat():
    jm = jnp.arange(_P_DIM * _Q_DIM, dtype=jnp.int32) // _Q_DIM
    rmat = (jm[None, :] == jnp.arange(_P_DIM, dtype=jnp.int32)[:, None])
    return rmat.astype(jnp.float32)


def _tc_expand(rows_x, w, v0):
    grid = _N // _TC_BLOCK
    return pl.pallas_call(
        _tc_expand_body,
        grid=(grid,),
        in_specs=[
            pl.BlockSpec((_TC_BLOCK, _P_DIM), lambda i: (i, 0)),
            pl.BlockSpec((_TC_BLOCK, 1), lambda i: (i, 0)),
            pl.BlockSpec((_Q_DIM, _Q_DIM), lambda i: (0, 0)),
            pl.BlockSpec((_P_DIM, _P_DIM * _Q_DIM), lambda i: (0, 0)),
        ],
        out_specs=pl.BlockSpec((_TC_BLOCK, _P_DIM * _Q_DIM), lambda i: (i, 0)),
        out_shape=jax.ShapeDtypeStruct((_N, _P_DIM * _Q_DIM), jnp.float32),
        compiler_params=pltpu.CompilerParams(
            dimension_semantics=("arbitrary",),
        ),
    )(rows_x, w.reshape(_N, 1), v0, _rmat())


@jax.jit
def kernel(d, w, x0, v0):
    rows_x = _sc_gather(x0, d)
    return _tc_expand(rows_x, w, v0)


# speedup vs baseline: 1.2665x; 1.0469x over previous
import functools

import jax
import jax.numpy as jnp
from jax import lax
from jax.experimental import pallas as pl
from jax.experimental.pallas import tpu as pltpu
from jax.experimental.pallas import tpu_sc as plsc

_N = 16384
_P_DIM = 64
_Q_DIM = 64
_NUM_WORKERS = 32
_RPW = _N // _NUM_WORKERS
_TC_BLOCK = 1024


def _sc_gather(x0, d):
    mesh = plsc.VectorSubcoreMesh(core_axis_name="c", subcore_axis_name="s")

    @functools.partial(
        pl.kernel,
        out_type=jax.ShapeDtypeStruct((_N, 128), jnp.float32),
        mesh=mesh,
        scratch_types=[
            pltpu.VMEM((_RPW,), jnp.int32),
            pltpu.VMEM((_RPW, _P_DIM), jnp.float32),
            pltpu.SemaphoreType.DMA,
        ],
        compiler_params=pltpu.CompilerParams(use_tc_tiling_on_sc=False),
    )
    def gather_kernel(x0_hbm, d_hbm, outx_hbm, idx_d, rows_x, sem_x):
        wid = lax.axis_index("s") * 2 + lax.axis_index("c")
        base = wid * _RPW
        pltpu.sync_copy(d_hbm.at[pl.ds(base, _RPW)], idx_d)
        pltpu.async_copy(x0_hbm.at[idx_d], rows_x, sem_x).wait()
        pltpu.sync_copy(rows_x, outx_hbm.at[pl.ds(base, _RPW), pl.ds(0, _P_DIM)])

    return gather_kernel(x0, d)


def _tc_expand_body(x_ref, w_ref, v_ref, r_ref, o_ref):
    x = x_ref[:, :_P_DIM]
    wi = w_ref[...]
    v = v_ref[...]
    sv = jnp.sum(v * v, axis=1, keepdims=True)
    vn = v * lax.rsqrt(sv)
    oh = (lax.broadcasted_iota(jnp.int32, (x.shape[0], _Q_DIM), 1) == wi)
    w_rows = jnp.dot(oh.astype(jnp.float32), vn,
                     preferred_element_type=jnp.float32)
    sx = jnp.sum(x * x, axis=1, keepdims=True)
    xs = x * lax.rsqrt(sx)
    xrep = jnp.dot(xs, r_ref[...], preferred_element_type=jnp.float32)
    wtile = pltpu.repeat(w_rows, _P_DIM, axis=1)
    o_ref[...] = xrep * wtile


def _rmat():
    jm = jnp.arange(_P_DIM * _Q_DIM, dtype=jnp.int32) // _Q_DIM
    rmat = (jm[None, :] == jnp.arange(_P_DIM, dtype=jnp.int32)[:, None])
    return rmat.astype(jnp.float32)


def _tc_expand(rows_x, w, v0):
    grid = _N // _TC_BLOCK
    return pl.pallas_call(
        _tc_expand_body,
        grid=(grid,),
        in_specs=[
            pl.BlockSpec((_TC_BLOCK, 128), lambda i: (i, 0)),
            pl.BlockSpec((_TC_BLOCK, 1), lambda i: (i, 0)),
            pl.BlockSpec((_Q_DIM, _Q_DIM), lambda i: (0, 0)),
            pl.BlockSpec((_P_DIM, _P_DIM * _Q_DIM), lambda i: (0, 0)),
        ],
        out_specs=pl.BlockSpec((_TC_BLOCK, _P_DIM * _Q_DIM), lambda i: (i, 0)),
        out_shape=jax.ShapeDtypeStruct((_N, _P_DIM * _Q_DIM), jnp.float32),
        compiler_params=pltpu.CompilerParams(
            dimension_semantics=("arbitrary",),
        ),
    )(rows_x, w.reshape(_N, 1), v0, _rmat())


@jax.jit
def kernel(d, w, x0, v0):
    rows_x = _sc_gather(x0, d)
    return _tc_expand(rows_x, w, v0)
